# baseline (speedup 1.0000x reference)
import functools

import jax
import jax.numpy as jnp
from jax import lax
from jax.experimental import pallas as pl
from jax.experimental.pallas import tpu as pltpu
from jax.experimental.pallas import tpu_sc as plsc

_VOCAB = 100000
_HIDDEN = 2048
_PLE_DIM = 256
_LAYER_IDX = 3
_COL0 = _LAYER_IDX * _PLE_DIM
_HALF = 128
_NC = 2
_NS = 16
_NW = _NC * _NS
_HPAD = 56
_NBUF = 4


@functools.lru_cache(maxsize=None)
def _make_gather(batch, hist):
    assert batch % _NW == 0
    b_per_w = batch // _NW
    n_idx = b_per_w * _HPAD
    assert (b_per_w - 8) % _NBUF == 0
    mesh = plsc.VectorSubcoreMesh(core_axis_name="c", subcore_axis_name="s")

    @functools.partial(
        pl.kernel,
        out_type=jax.ShapeDtypeStruct((batch, hist, _PLE_DIM), jnp.float32),
        mesh=mesh,
        scratch_types=[
            pltpu.VMEM((n_idx,), jnp.int32),
        ] + [pltpu.VMEM((hist, _PLE_DIM), jnp.float32)] * _NBUF
          + [pltpu.SemaphoreType.DMA] * (2 * _NBUF),
    )
    def k(tab_hbm, idx_hbm, out_hbm, idx_v, *bufs_sems):
        bufs = bufs_sems[:_NBUF]
        gsems = bufs_sems[_NBUF:2 * _NBUF]
        wsems = bufs_sems[2 * _NBUF:]
        wid = lax.axis_index("s") * _NC + lax.axis_index("c")
        base = wid * b_per_w
        pltpu.sync_copy(idx_hbm.at[pl.ds(wid * n_idx, n_idx)], idx_v)

        def gather(c, s):
            idx_c = idx_v.at[pl.ds(c * _HPAD, hist)]
            pltpu.async_copy(
                tab_hbm.at[idx_c, pl.ds(_COL0, _HALF)],
                bufs[s].at[:, pl.ds(0, _HALF)], gsems[s])
            pltpu.async_copy(
                tab_hbm.at[idx_c, pl.ds(_COL0 + _HALF, _HALF)],
                bufs[s].at[:, pl.ds(_HALF, _HALF)], gsems[s])

        def write(c, s):
            pltpu.async_copy(bufs[s], out_hbm.at[base + c], wsems[s])

        def wait_g(s):
            idx_c = idx_v.at[pl.ds(0, hist)]
            pltpu.make_async_copy(
                tab_hbm.at[idx_c, pl.ds(_COL0, _HALF)],
                bufs[s].at[:, pl.ds(0, _HALF)], gsems[s]).wait()
            pltpu.make_async_copy(
                tab_hbm.at[idx_c, pl.ds(_COL0 + _HALF, _HALF)],
                bufs[s].at[:, pl.ds(_HALF, _HALF)], gsems[s]).wait()

        def wait_w(s):
            pltpu.make_async_copy(bufs[s], out_hbm.at[base], wsems[s]).wait()

        gather(0, 0)
        gather(1, 1)
        for c in range(2, 8):
            if c >= 4:
                wait_w(c % _NBUF)
            gather(c, c % _NBUF)
            wait_g((c - 2) % _NBUF)
            write(c - 2, (c - 2) % _NBUF)

        def body(i, _):
            for j in range(_NBUF):
                c = 8 + _NBUF * i + j
                s = j
                s2 = (2 + j) % _NBUF
                wait_w(s)
                gather(c, s)
                wait_g(s2)
                write(c - 2, s2)
            return 0

        lax.fori_loop(0, (b_per_w - 8) // _NBUF, body, 0)

        for c in range(b_per_w - 2, b_per_w):
            s = c % _NBUF
            wait_g(s)
            write(c, s)
        for s in range(_NBUF):
            wait_w(s)

    return k


def kernel(input_ids, table):
    batch, hist = input_ids.shape
    ids = jnp.pad(input_ids.astype(jnp.int32),
                  ((0, 0), (0, _HPAD - hist))).reshape(batch * _HPAD)
    half = batch // 2
    gath = _make_gather(half, hist)
    o1 = gath(table, ids[: half * _HPAD])
    o2 = gath(table, ids[half * _HPAD:])
    return jnp.concatenate([o1, o2], axis=0)


# speedup vs baseline: 1.4915x; 1.4915x over previous
import functools

import jax
import jax.numpy as jnp
from jax import lax
from jax.experimental import pallas as pl
from jax.experimental.pallas import tpu as pltpu
from jax.experimental.pallas import tpu_sc as plsc

_VOCAB = 100000
_HIDDEN = 2048
_PLE_DIM = 256
_LAYER_IDX = 3
_COL0 = _LAYER_IDX * _PLE_DIM
_HALF = 128
_NC = 2
_NS = 16
_NW = _NC * _NS
_HPAD = 56
_NBUF = 4


@functools.lru_cache(maxsize=None)
def _make_gather(batch, hist):
    assert batch % _NW == 0
    b_per_w = batch // _NW
    n_idx = b_per_w * _HPAD
    assert (b_per_w - 8) % _NBUF == 0
    mesh = plsc.VectorSubcoreMesh(core_axis_name="c", subcore_axis_name="s")

    @functools.partial(
        pl.kernel,
        out_type=jax.ShapeDtypeStruct((batch, hist, _PLE_DIM), jnp.float32),
        mesh=mesh,
        compiler_params=pltpu.CompilerParams(use_tc_tiling_on_sc=True),
        scratch_types=[
            pltpu.VMEM((n_idx,), jnp.int32),
        ] + [pltpu.VMEM((hist, _PLE_DIM), jnp.float32)] * _NBUF
          + [pltpu.SemaphoreType.DMA] * (2 * _NBUF),
    )
    def k(tab_hbm, idx_hbm, out_hbm, idx_v, *bufs_sems):
        bufs = bufs_sems[:_NBUF]
        gsems = bufs_sems[_NBUF:2 * _NBUF]
        wsems = bufs_sems[2 * _NBUF:]
        wid = lax.axis_index("s") * _NC + lax.axis_index("c")
        base = wid * b_per_w
        pltpu.sync_copy(idx_hbm.at[pl.ds(wid * n_idx, n_idx)], idx_v)

        def gather(c, s):
            idx_c = idx_v.at[pl.ds(c * _HPAD, hist)]
            pltpu.async_copy(
                tab_hbm.at[idx_c, pl.ds(_COL0, _HALF)],
                bufs[s].at[:, pl.ds(0, _HALF)], gsems[s])
            pltpu.async_copy(
                tab_hbm.at[idx_c, pl.ds(_COL0 + _HALF, _HALF)],
                bufs[s].at[:, pl.ds(_HALF, _HALF)], gsems[s])

        def write(c, s):
            pltpu.async_copy(bufs[s], out_hbm.at[base + c], wsems[s])

        def wait_g(s):
            idx_c = idx_v.at[pl.ds(0, hist)]
            pltpu.make_async_copy(
                tab_hbm.at[idx_c, pl.ds(_COL0, _HALF)],
                bufs[s].at[:, pl.ds(0, _HALF)], gsems[s]).wait()
            pltpu.make_async_copy(
                tab_hbm.at[idx_c, pl.ds(_COL0 + _HALF, _HALF)],
                bufs[s].at[:, pl.ds(_HALF, _HALF)], gsems[s]).wait()

        def wait_w(s):
            pltpu.make_async_copy(bufs[s], out_hbm.at[base], wsems[s]).wait()

        gather(0, 0)
        gather(1, 1)
        for c in range(2, 8):
            if c >= 4:
                wait_w(c % _NBUF)
            gather(c, c % _NBUF)
            wait_g((c - 2) % _NBUF)
            write(c - 2, (c - 2) % _NBUF)

        def body(i, _):
            for j in range(_NBUF):
                c = 8 + _NBUF * i + j
                s = j
                s2 = (2 + j) % _NBUF
                wait_w(s)
                gather(c, s)
                wait_g(s2)
                write(c - 2, s2)
            return 0

        lax.fori_loop(0, (b_per_w - 8) // _NBUF, body, 0)

        for c in range(b_per_w - 2, b_per_w):
            s = c % _NBUF
            wait_g(s)
            write(c, s)
        for s in range(_NBUF):
            wait_w(s)

    return k


def kernel(input_ids, table):
    batch, hist = input_ids.shape
    ids = jnp.pad(input_ids.astype(jnp.int32),
                  ((0, 0), (0, _HPAD - hist))).reshape(batch * _HPAD)
    return _make_gather(batch, hist)(table, ids)


# speedup vs baseline: 1.4952x; 1.0025x over previous
import functools

import jax
import jax.numpy as jnp
from jax import lax
from jax.experimental import pallas as pl
from jax.experimental.pallas import tpu as pltpu
from jax.experimental.pallas import tpu_sc as plsc

_VOCAB = 100000
_HIDDEN = 2048
_PLE_DIM = 256
_LAYER_IDX = 3
_COL0 = _LAYER_IDX * _PLE_DIM
_HALF = 128
_NC = 2
_NS = 16
_NW = _NC * _NS
_HPAD = 56
_EPC = 4


@functools.lru_cache(maxsize=None)
def _make_gather(batch, hist):
    assert batch % (_NW * _EPC) == 0
    b_per_w = batch // _NW
    n_idx = b_per_w * _HPAD
    n_chunks = b_per_w // _EPC
    mesh = plsc.VectorSubcoreMesh(core_axis_name="c", subcore_axis_name="s")

    @functools.partial(
        pl.kernel,
        out_type=jax.ShapeDtypeStruct((batch, hist, _PLE_DIM), jnp.float32),
        mesh=mesh,
        scratch_types=[
            pltpu.VMEM((n_idx,), jnp.int32),
            pltpu.VMEM((_EPC, hist, _PLE_DIM), jnp.float32),
            pltpu.VMEM((_EPC, hist, _PLE_DIM), jnp.float32),
            pltpu.SemaphoreType.DMA,
            pltpu.SemaphoreType.DMA,
            pltpu.SemaphoreType.DMA,
            pltpu.SemaphoreType.DMA,
        ],
    )
    def k(tab_hbm, idx_hbm, out_hbm, idx_v, buf0, buf1, gs0, gs1, ws0, ws1):
        wid = lax.axis_index("s") * _NC + lax.axis_index("c")
        base = wid * b_per_w
        pltpu.sync_copy(idx_hbm.at[pl.ds(wid * n_idx, n_idx)], idx_v)

        bufs = (buf0, buf1)
        gsems = (gs0, gs1)
        wsems = (ws0, ws1)

        def gather(c, b):
            for e in range(_EPC):
                idx_c = idx_v.at[pl.ds((c * _EPC + e) * _HPAD, hist)]
                pltpu.async_copy(
                    tab_hbm.at[idx_c, pl.ds(_COL0, _HALF)],
                    bufs[b].at[e, :, pl.ds(0, _HALF)], gsems[b])
                pltpu.async_copy(
                    tab_hbm.at[idx_c, pl.ds(_COL0 + _HALF, _HALF)],
                    bufs[b].at[e, :, pl.ds(_HALF, _HALF)], gsems[b])

        def write(c, b):
            pltpu.async_copy(
                bufs[b], out_hbm.at[pl.ds(base + c * _EPC, _EPC)], wsems[b])

        def wait_g(b):
            idx_c = idx_v.at[pl.ds(0, hist)]
            for e in range(_EPC):
                pltpu.make_async_copy(
                    tab_hbm.at[idx_c, pl.ds(_COL0, _HALF)],
                    bufs[b].at[e, :, pl.ds(0, _HALF)], gsems[b]).wait()
                pltpu.make_async_copy(
                    tab_hbm.at[idx_c, pl.ds(_COL0 + _HALF, _HALF)],
                    bufs[b].at[e, :, pl.ds(_HALF, _HALF)], gsems[b]).wait()

        def wait_w(b):
            pltpu.make_async_copy(
                bufs[b], out_hbm.at[pl.ds(base, _EPC)], wsems[b]).wait()

        gather(0, 0)
        gather(1, 1)

        def body(i, _):
            for b in range(2):
                c = 2 * i + b
                wait_g(b)
                write(c, b)
                wait_w(b)
                gather(c + 2, b)
            return 0

        lax.fori_loop(0, n_chunks // 2 - 1, body, 0)

        for b in range(2):
            c = n_chunks - 2 + b
            wait_g(b)
            write(c, b)
        wait_w(0)
        wait_w(1)

    return k


def kernel(input_ids, table):
    batch, hist = input_ids.shape
    ids = jnp.pad(input_ids.astype(jnp.int32),
                  ((0, 0), (0, _HPAD - hist))).reshape(batch * _HPAD)
    return _make_gather(batch, hist)(table, ids)


# speedup vs baseline: 1.4995x; 1.0029x over previous
import functools

import jax
import jax.numpy as jnp
from jax import lax
from jax.experimental import pallas as pl
from jax.experimental.pallas import tpu as pltpu
from jax.experimental.pallas import tpu_sc as plsc

_VOCAB = 100000
_HIDDEN = 2048
_PLE_DIM = 256
_LAYER_IDX = 3
_COL0 = _LAYER_IDX * _PLE_DIM
_HALF = 128
_NC = 2
_NS = 16
_NW = _NC * _NS
_HPAD = 56
_EPC = 4


@functools.lru_cache(maxsize=None)
def _make_gather(batch, hist):
    assert batch % (_NW * _EPC) == 0
    b_per_w = batch // _NW
    n_idx = b_per_w * _HPAD
    n_chunks = b_per_w // _EPC
    mesh = plsc.VectorSubcoreMesh(core_axis_name="c", subcore_axis_name="s")

    @functools.partial(
        pl.kernel,
        out_type=jax.ShapeDtypeStruct((batch, hist, _PLE_DIM), jnp.float32),
        mesh=mesh,
        compiler_params=pltpu.CompilerParams(needs_layout_passes=True),
        scratch_types=[
            pltpu.VMEM((n_idx,), jnp.int32),
            pltpu.VMEM((_EPC, hist, _PLE_DIM), jnp.float32),
            pltpu.VMEM((_EPC, hist, _PLE_DIM), jnp.float32),
            pltpu.SemaphoreType.DMA,
            pltpu.SemaphoreType.DMA,
            pltpu.SemaphoreType.DMA,
            pltpu.SemaphoreType.DMA,
        ],
    )
    def k(tab_hbm, idx_hbm, out_hbm, idx_v, buf0, buf1, gs0, gs1, ws0, ws1):
        wid = lax.axis_index("s") * _NC + lax.axis_index("c")
        base = wid * b_per_w
        pltpu.sync_copy(idx_hbm.at[pl.ds(wid * n_idx, n_idx)], idx_v)

        bufs = (buf0, buf1)
        gsems = (gs0, gs1)
        wsems = (ws0, ws1)

        def gather(c, b):
            for e in range(_EPC):
                idx_c = idx_v.at[pl.ds((c * _EPC + e) * _HPAD, hist)]
                pltpu.async_copy(
                    tab_hbm.at[idx_c, pl.ds(_COL0, _HALF)],
                    bufs[b].at[e, :, pl.ds(0, _HALF)], gsems[b])
                pltpu.async_copy(
                    tab_hbm.at[idx_c, pl.ds(_COL0 + _HALF, _HALF)],
                    bufs[b].at[e, :, pl.ds(_HALF, _HALF)], gsems[b])

        def write(c, b):
            pltpu.async_copy(
                bufs[b], out_hbm.at[pl.ds(base + c * _EPC, _EPC)], wsems[b])

        def wait_g(b):
            idx_c = idx_v.at[pl.ds(0, hist)]
            for e in range(_EPC):
                pltpu.make_async_copy(
                    tab_hbm.at[idx_c, pl.ds(_COL0, _HALF)],
                    bufs[b].at[e, :, pl.ds(0, _HALF)], gsems[b]).wait()
                pltpu.make_async_copy(
                    tab_hbm.at[idx_c, pl.ds(_COL0 + _HALF, _HALF)],
                    bufs[b].at[e, :, pl.ds(_HALF, _HALF)], gsems[b]).wait()

        def wait_w(b):
            pltpu.make_async_copy(
                bufs[b], out_hbm.at[pl.ds(base, _EPC)], wsems[b]).wait()

        gather(0, 0)
        gather(1, 1)

        def body(i, _):
            for b in range(2):
                c = 2 * i + b
                wait_g(b)
                write(c, b)
                wait_w(b)
                gather(c + 2, b)
            return 0

        lax.fori_loop(0, n_chunks // 2 - 1, body, 0)

        for b in range(2):
            c = n_chunks - 2 + b
            wait_g(b)
            write(c, b)
        wait_w(0)
        wait_w(1)

    return k


def kernel(input_ids, table):
    batch, hist = input_ids.shape
    ids = jnp.pad(input_ids.astype(jnp.int32),
                  ((0, 0), (0, _HPAD - hist))).reshape(batch * _HPAD)
    return _make_gather(batch, hist)(table, ids)
